# Initial kernel scaffold; baseline (speedup 1.0000x reference)
import jax
import jax.numpy as jnp
from jax.experimental import pallas as pl


def kernel(x, edge_index, W0, b0, We1, be1, We2, be2, Wroot, bconv, W1, b1, W2, b2):
    raise NotImplementedError("write your pallas kernel here")


# speedup vs baseline: 4.0947x; 4.0947x over previous
import functools

import jax
import jax.numpy as jnp
from jax import lax
from jax.experimental import pallas as pl
from jax.experimental.pallas import tpu as pltpu
from jax.experimental.pallas import tpu_sc as plsc

NC = 2
NS = 16
NW = NC * NS


def _tc1_body(x_ref, w0_ref, b0_ref, h_ref):
    h_ref[...] = jax.nn.relu(
        jnp.dot(x_ref[...], w0_ref[...], preferred_element_type=jnp.float32,
                precision=lax.Precision.HIGHEST)
        + b0_ref[...]
    )


def _tc2_body(h_ref, p_ref, wroot_ref, we_ref, bconv_ref, out_ref):
    a = p_ref[0] + p_ref[1]
    out_ref[...] = (
        jnp.dot(h_ref[...], wroot_ref[...], preferred_element_type=jnp.float32,
                precision=lax.Precision.HIGHEST)
        + jnp.dot(a, we_ref[...], preferred_element_type=jnp.float32,
                precision=lax.Precision.HIGHEST)
        + bconv_ref[...]
    )


def _tc3_body(emb_ref, w1_ref, b1_ref, w2_ref, b2_ref, out_ref):
    ee = jax.nn.relu(
        jnp.dot(emb_ref[...], w1_ref[...], preferred_element_type=jnp.float32,
                precision=lax.Precision.HIGHEST)
        + b1_ref[...]
    )
    out_ref[...] = (
        jnp.dot(ee, w2_ref[...], preferred_element_type=jnp.float32,
                precision=lax.Precision.HIGHEST)
        + b2_ref[...]
    )


CHUNK = 128


def _stage_indices(src2_hbm, dst2_hbm, sidx_v, didx_v, wid, per_w, extra):
    base = wid * per_w
    pltpu.sync_copy(src2_hbm.at[pl.ds(base, per_w)], sidx_v.at[pl.ds(0, per_w)])
    pltpu.sync_copy(dst2_hbm.at[pl.ds(base, per_w)], didx_v.at[pl.ds(0, per_w)])

    @pl.when(wid < extra)
    def _tail():
        t = per_w * NW + wid
        pltpu.sync_copy(src2_hbm.at[pl.ds(t, 1)], sidx_v.at[pl.ds(per_w, 1)])
        pltpu.sync_copy(dst2_hbm.at[pl.ds(t, 1)], didx_v.at[pl.ds(per_w, 1)])


def _sc_segsum_body(h_hbm, src2_hbm, dst2_hbm, part_hbm,
                    sidx_v, didx_v, rows_v, zrow_v, acc_sh, sem,
                    *, per_w, extra, rows_per_sub):
    cid = lax.axis_index("c")
    sid = lax.axis_index("s")
    wid = sid * NC + cid

    zchunk = zrow_v.shape[0]

    @pl.loop(0, zchunk)
    def _zero_buf(i):
        zrow_v[i, :] = jnp.zeros((16,), jnp.float32)

    @pl.loop(0, rows_per_sub // zchunk)
    def _zero_acc(k):
        pltpu.sync_copy(zrow_v, acc_sh.at[pl.ds(sid * rows_per_sub + k * zchunk, zchunk)])

    _stage_indices(src2_hbm, dst2_hbm, sidx_v, didx_v, wid, per_w, extra)
    plsc.subcore_barrier()

    nch = per_w + jnp.where(wid < extra, 1, 0)

    @pl.loop(0, nch)
    def _edges(j):
        pltpu.async_copy(h_hbm.at[sidx_v.at[j]], rows_v, sem).wait()
        pltpu.sync_copy(rows_v, acc_sh.at[didx_v.at[j]], add=True)

    plsc.subcore_barrier()

    pltpu.sync_copy(acc_sh.at[pl.ds(sid * rows_per_sub, rows_per_sub)],
                    part_hbm.at[cid, pl.ds(sid * rows_per_sub, rows_per_sub)])


def _sc_edgemul_body(out_hbm, src2_hbm, dst2_hbm, emb_hbm,
                     sidx_v, didx_v, srows_v, drows_v, sem, sem2,
                     *, per_w, extra):
    cid = lax.axis_index("c")
    sid = lax.axis_index("s")
    wid = sid * NC + cid

    _stage_indices(src2_hbm, dst2_hbm, sidx_v, didx_v, wid, per_w, extra)

    nch = per_w + jnp.where(wid < extra, 1, 0)

    @pl.loop(0, nch)
    def _edges(j):
        cp1 = pltpu.async_copy(out_hbm.at[sidx_v.at[j]], srows_v, sem)
        cp2 = pltpu.async_copy(out_hbm.at[didx_v.at[j]], drows_v, sem2)
        cp1.wait()
        cp2.wait()

        @pl.loop(0, CHUNK, unroll=8)
        def _mul(r):
            srows_v[r, :] = srows_v[r, :] * drows_v[r, :]

        chunk_id = jnp.where(j == per_w, per_w * NW + wid, wid * per_w + j)
        pltpu.sync_copy(srows_v, emb_hbm.at[pl.ds(chunk_id * CHUNK, CHUNK)])


def kernel(x, edge_index, W0, b0, We1, be1, We2, be2, Wroot, bconv, W1, b1,
           W2, b2):
    n, d = x.shape
    e = edge_index.shape[1]
    h_dim = W0.shape[1]

    src = edge_index[0]
    dst = edge_index[1]

    e1 = jax.nn.relu(We1[0] + be1)
    w_e = (e1 @ We2 + be2).reshape(h_dim, h_dim)

    h = pl.pallas_call(
        _tc1_body,
        out_shape=jax.ShapeDtypeStruct((n, h_dim), jnp.float32),
    )(x, W0, b0.reshape(1, h_dim))

    chunks = e // CHUNK
    per_w = chunks // NW
    extra = chunks - per_w * NW
    src2 = src.reshape(chunks, CHUNK)
    dst2 = dst.reshape(chunks, CHUNK)
    rows_per_sub = -(-n // (8 * NS)) * 8
    npad = rows_per_sub * NS
    zchunk = rows_per_sub // 4
    mesh = plsc.VectorSubcoreMesh(core_axis_name="c", subcore_axis_name="s",
                                  num_cores=NC, num_subcores=NS)
    seg = functools.partial(_sc_segsum_body, per_w=per_w, extra=extra,
                            rows_per_sub=rows_per_sub)
    partials = pl.kernel(
        seg,
        out_type=jax.ShapeDtypeStruct((NC, npad, h_dim), jnp.float32),
        mesh=mesh,
        scratch_types=[
            pltpu.VMEM((per_w + 1, CHUNK), jnp.int32),
            pltpu.VMEM((per_w + 1, CHUNK), jnp.int32),
            pltpu.VMEM((CHUNK, h_dim), jnp.float32),
            pltpu.VMEM((zchunk, h_dim), jnp.float32),
            pltpu.VMEM_SHARED((npad, h_dim), jnp.float32),
            pltpu.SemaphoreType.DMA,
        ],
        compiler_params=pltpu.CompilerParams(use_tc_tiling_on_sc=False),
    )(h, src2, dst2)
    partials = partials[:, :n]

    out = pl.pallas_call(
        _tc2_body,
        out_shape=jax.ShapeDtypeStruct((n, h_dim), jnp.float32),
    )(h, partials, Wroot, w_e, bconv.reshape(1, h_dim))

    mul = functools.partial(_sc_edgemul_body, per_w=per_w, extra=extra)
    emb = pl.kernel(
        mul,
        out_type=jax.ShapeDtypeStruct((e, h_dim), jnp.float32),
        mesh=mesh,
        scratch_types=[
            pltpu.VMEM((per_w + 1, CHUNK), jnp.int32),
            pltpu.VMEM((per_w + 1, CHUNK), jnp.int32),
            pltpu.VMEM((CHUNK, h_dim), jnp.float32),
            pltpu.VMEM((CHUNK, h_dim), jnp.float32),
            pltpu.SemaphoreType.DMA,
            pltpu.SemaphoreType.DMA,
        ],
        compiler_params=pltpu.CompilerParams(use_tc_tiling_on_sc=False),
    )(out, src2, dst2)

    blk = 8000
    score = pl.pallas_call(
        _tc3_body,
        grid=(e // blk,),
        in_specs=[
            pl.BlockSpec((blk, h_dim), lambda i: (i, 0)),
            pl.BlockSpec((h_dim, 8), lambda i: (0, 0)),
            pl.BlockSpec((1, 8), lambda i: (0, 0)),
            pl.BlockSpec((8, 1), lambda i: (0, 0)),
            pl.BlockSpec((1, 1), lambda i: (0, 0)),
        ],
        out_specs=pl.BlockSpec((blk, 1), lambda i: (i, 0)),
        out_shape=jax.ShapeDtypeStruct((e, 1), jnp.float32),
    )(emb, W1, b1.reshape(1, 8), W2, b2.reshape(1, 1))

    return score.reshape(-1)


# speedup vs baseline: 11.0993x; 2.7107x over previous
import functools

import jax
import jax.numpy as jnp
from jax import lax
from jax.experimental import pallas as pl
from jax.experimental.pallas import tpu as pltpu
from jax.experimental.pallas import tpu_sc as plsc

NC = 2
NS = 16
NW = NC * NS


def _tc1_body(x_ref, w0_ref, b0_ref, h_ref):
    h_ref[...] = jax.nn.relu(
        jnp.dot(x_ref[...], w0_ref[...], preferred_element_type=jnp.float32,
                precision=lax.Precision.HIGHEST)
        + b0_ref[...]
    )


def _tc2_body(h_ref, p_ref, wroot_ref, we_ref, bconv_ref, out_ref):
    a = p_ref[0] + p_ref[1]
    out_ref[...] = (
        jnp.dot(h_ref[...], wroot_ref[...], preferred_element_type=jnp.float32,
                precision=lax.Precision.HIGHEST)
        + jnp.dot(a, we_ref[...], preferred_element_type=jnp.float32,
                precision=lax.Precision.HIGHEST)
        + bconv_ref[...]
    )


def _tc3_body(emb_ref, w1_ref, b1_ref, w2_ref, b2_ref, out_ref):
    ee = jax.nn.relu(
        jnp.dot(emb_ref[...], w1_ref[...], preferred_element_type=jnp.float32,
                precision=lax.Precision.HIGHEST)
        + b1_ref[...]
    )
    out_ref[...] = (
        jnp.dot(ee, w2_ref[...], preferred_element_type=jnp.float32,
                precision=lax.Precision.HIGHEST)
        + b2_ref[...]
    )


CHUNK = 128


def _stage_indices(src2_hbm, dst2_hbm, sidx_v, didx_v, wid, per_w, extra):
    base = wid * per_w
    pltpu.sync_copy(src2_hbm.at[pl.ds(base, per_w)], sidx_v.at[pl.ds(0, per_w)])
    pltpu.sync_copy(dst2_hbm.at[pl.ds(base, per_w)], didx_v.at[pl.ds(0, per_w)])

    @pl.when(wid < extra)
    def _tail():
        t = per_w * NW + wid
        pltpu.sync_copy(src2_hbm.at[pl.ds(t, 1)], sidx_v.at[pl.ds(per_w, 1)])
        pltpu.sync_copy(dst2_hbm.at[pl.ds(t, 1)], didx_v.at[pl.ds(per_w, 1)])


def _sc_segsum_body(h_hbm, src2_hbm, dst2_hbm, part_hbm,
                    sidx_v, didx_v, rows_v, zrow_v, acc_sh, sem,
                    *, per_w, extra, rows_per_sub, rows_last):
    cid = lax.axis_index("c")
    sid = lax.axis_index("s")
    wid = sid * NC + cid

    zchunk = zrow_v.shape[0]

    @pl.loop(0, zchunk)
    def _zero_buf(i):
        zrow_v[i, :] = jnp.zeros((16,), jnp.float32)

    @pl.loop(0, rows_per_sub // zchunk)
    def _zero_acc(k):
        pltpu.sync_copy(zrow_v, acc_sh.at[pl.ds(sid * rows_per_sub + k * zchunk, zchunk)])

    _stage_indices(src2_hbm, dst2_hbm, sidx_v, didx_v, wid, per_w, extra)
    plsc.subcore_barrier()

    nch = per_w + jnp.where(wid < extra, 1, 0)

    @pl.loop(0, nch)
    def _edges(j):
        pltpu.async_copy(h_hbm.at[sidx_v.at[j]], rows_v, sem).wait()
        pltpu.sync_copy(rows_v, acc_sh.at[didx_v.at[j]], add=True)

    plsc.subcore_barrier()

    @pl.when(sid < NS - 1)
    def _full():
        pltpu.sync_copy(acc_sh.at[pl.ds(sid * rows_per_sub, rows_per_sub)],
                        part_hbm.at[cid, pl.ds(sid * rows_per_sub, rows_per_sub)])

    @pl.when(sid == NS - 1)
    def _clipped():
        pltpu.sync_copy(acc_sh.at[pl.ds((NS - 1) * rows_per_sub, rows_last)],
                        part_hbm.at[cid, pl.ds((NS - 1) * rows_per_sub, rows_last)])


def _sc_edgemul_body(out_hbm, src2_hbm, dst2_hbm, emb_hbm,
                     sidx_v, didx_v, srows_v, drows_v, sem, sem2,
                     *, per_w, extra):
    cid = lax.axis_index("c")
    sid = lax.axis_index("s")
    wid = sid * NC + cid

    _stage_indices(src2_hbm, dst2_hbm, sidx_v, didx_v, wid, per_w, extra)

    nch = per_w + jnp.where(wid < extra, 1, 0)

    @pl.loop(0, nch)
    def _edges(j):
        cp1 = pltpu.async_copy(out_hbm.at[sidx_v.at[j]], srows_v, sem)
        cp2 = pltpu.async_copy(out_hbm.at[didx_v.at[j]], drows_v, sem2)
        cp1.wait()
        cp2.wait()

        @pl.loop(0, CHUNK, unroll=8)
        def _mul(r):
            srows_v[r, :] = srows_v[r, :] * drows_v[r, :]

        chunk_id = jnp.where(j == per_w, per_w * NW + wid, wid * per_w + j)
        pltpu.sync_copy(srows_v, emb_hbm.at[pl.ds(chunk_id * CHUNK, CHUNK)])


def kernel(x, edge_index, W0, b0, We1, be1, We2, be2, Wroot, bconv, W1, b1,
           W2, b2):
    n, d = x.shape
    e = edge_index.shape[1]
    h_dim = W0.shape[1]

    src = edge_index[0]
    dst = edge_index[1]

    e1 = jax.nn.relu(We1[0] + be1)
    w_e = (e1 @ We2 + be2).reshape(h_dim, h_dim)

    P = 128 // h_dim
    eyeP = jnp.eye(P, dtype=jnp.float32)
    W0big = jnp.kron(eyeP, W0)
    b0big = jnp.tile(b0, P).reshape(1, P * h_dim)
    Wrootbig = jnp.kron(eyeP, Wroot)
    Webig = jnp.kron(eyeP, w_e)
    bconvbig = jnp.tile(bconv, P).reshape(1, P * h_dim)
    W1big = jnp.kron(eyeP, W1)
    b1big = jnp.tile(b1, P).reshape(1, P * 8)
    W2big = jnp.kron(eyeP, W2)

    h_p = pl.pallas_call(
        _tc1_body,
        out_shape=jax.ShapeDtypeStruct((n // P, P * h_dim), jnp.float32),
    )(x.reshape(n // P, P * d), W0big, b0big)
    h = h_p.reshape(n, h_dim)

    chunks = e // CHUNK
    per_w = chunks // NW
    extra = chunks - per_w * NW
    src2 = src.reshape(chunks, CHUNK)
    dst2 = dst.reshape(chunks, CHUNK)
    rows_per_sub = -(-n // (8 * NS)) * 8
    npad = rows_per_sub * NS
    zchunk = rows_per_sub // 4
    mesh = plsc.VectorSubcoreMesh(core_axis_name="c", subcore_axis_name="s",
                                  num_cores=NC, num_subcores=NS)
    rows_last = n - rows_per_sub * (NS - 1)
    seg = functools.partial(_sc_segsum_body, per_w=per_w, extra=extra,
                            rows_per_sub=rows_per_sub, rows_last=rows_last)
    partials = pl.kernel(
        seg,
        out_type=jax.ShapeDtypeStruct((NC, n, h_dim), jnp.float32),
        mesh=mesh,
        scratch_types=[
            pltpu.VMEM((per_w + 1, CHUNK), jnp.int32),
            pltpu.VMEM((per_w + 1, CHUNK), jnp.int32),
            pltpu.VMEM((CHUNK, h_dim), jnp.float32),
            pltpu.VMEM((zchunk, h_dim), jnp.float32),
            pltpu.VMEM_SHARED((npad, h_dim), jnp.float32),
            pltpu.SemaphoreType.DMA,
        ],
        compiler_params=pltpu.CompilerParams(use_tc_tiling_on_sc=False),
    )(h, src2, dst2)

    part_p = partials.reshape(NC, n // P, P * h_dim)
    out_p = pl.pallas_call(
        _tc2_body,
        out_shape=jax.ShapeDtypeStruct((n // P, P * h_dim), jnp.float32),
    )(h_p, part_p, Wrootbig, Webig, bconvbig)
    out = out_p.reshape(n, h_dim)

    mul = functools.partial(_sc_edgemul_body, per_w=per_w, extra=extra)
    emb = pl.kernel(
        mul,
        out_type=jax.ShapeDtypeStruct((e, h_dim), jnp.float32),
        mesh=mesh,
        scratch_types=[
            pltpu.VMEM((per_w + 1, CHUNK), jnp.int32),
            pltpu.VMEM((per_w + 1, CHUNK), jnp.int32),
            pltpu.VMEM((CHUNK, h_dim), jnp.float32),
            pltpu.VMEM((CHUNK, h_dim), jnp.float32),
            pltpu.SemaphoreType.DMA,
            pltpu.SemaphoreType.DMA,
        ],
        compiler_params=pltpu.CompilerParams(use_tc_tiling_on_sc=False),
    )(out, src2, dst2)

    emb_p = emb.reshape(e // P, P * h_dim)
    blk = 4000
    score = pl.pallas_call(
        _tc3_body,
        grid=(e // P // blk,),
        in_specs=[
            pl.BlockSpec((blk, P * h_dim), lambda i: (i, 0)),
            pl.BlockSpec((P * h_dim, P * 8), lambda i: (0, 0)),
            pl.BlockSpec((1, P * 8), lambda i: (0, 0)),
            pl.BlockSpec((P * 8, P), lambda i: (0, 0)),
            pl.BlockSpec((1, 1), lambda i: (0, 0)),
        ],
        out_specs=pl.BlockSpec((blk, P), lambda i: (i, 0)),
        out_shape=jax.ShapeDtypeStruct((e // P, P), jnp.float32),
    )(emb_p, W1big, b1big, W2big, b2.reshape(1, 1))

    return score.reshape(-1)
